# baseline (speedup 1.0000x reference)
import functools

import jax
import jax.numpy as jnp
from jax import lax
from jax.experimental import pallas as pl
from jax.experimental.pallas import tpu as pltpu
from jax.experimental.pallas import tpu_sc as plsc

B_, S, D, E, F, K = 1, 2048, 1024, 8, 512, 2
T = B_ * S
TB = 128
NP = T * K // TB + E
NROWS = NP * TB
NC, NS = 2, 16
NW = NC * NS
TPW = T // NW

_f32 = jnp.float32
_i32 = jnp.int32


def _route_body(x_ref, rw_ref, tw0_ref, tw1_ref, inv0_ref, inv1_ref,
                be_ref, meta_ref, loss_ref):
    x = x_ref[...]
    logits = jnp.dot(x, rw_ref[...], preferred_element_type=_f32)
    probs = jax.nn.softmax(logits, axis=-1)
    i1 = jnp.argmax(probs, axis=-1)
    m1 = jnp.max(probs, axis=-1, keepdims=True)
    iota_e = lax.broadcasted_iota(_i32, probs.shape, 1)
    oh1 = iota_e == i1[:, None]
    masked = jnp.where(oh1, -jnp.inf, probs)
    i2 = jnp.argmax(masked, axis=-1)
    m2 = jnp.max(masked, axis=-1, keepdims=True)
    oh2 = iota_e == i2[:, None]
    denom = m1 + m2
    tw0_ref[...] = jnp.broadcast_to(m1 / denom, (T, 128))
    tw1_ref[...] = jnp.broadcast_to(m2 / denom, (T, 128))
    mask = (oh1 | oh2).astype(_f32)
    loss_ref[...] = (E * jnp.sum(jnp.mean(mask, axis=0)
                                 * jnp.mean(probs, axis=0))).reshape(1, 1)

    maskT = jnp.transpose(mask)
    incl = maskT
    sh = 1
    while sh < T:
        incl = incl + jnp.concatenate(
            [jnp.zeros((E, sh), _f32), incl[:, :T - sh]], axis=1)
        sh *= 2
    posE = incl - maskT
    counts = incl[:, T - 1:T]

    pc = ((counts.astype(_i32) + (TB - 1)) // TB) * TB
    r8 = lax.broadcasted_iota(_i32, (E, E), 0)
    c8 = lax.broadcasted_iota(_i32, (E, E), 1)
    tri8 = (r8 > c8).astype(_f32)
    poff = jnp.dot(tri8, pc.astype(_f32), preferred_element_type=_f32)
    total_pad = jnp.sum(pc)

    inv_all = poff + posE
    ohT1 = jnp.transpose(oh1.astype(_f32))
    ohT2 = jnp.transpose(oh2.astype(_f32))
    inv0_ref[...] = jnp.sum(ohT1 * inv_all, axis=0, keepdims=True).astype(_i32)
    inv1_ref[...] = jnp.sum(ohT2 * inv_all, axis=0, keepdims=True).astype(_i32)

    bt = jnp.minimum(lax.broadcasted_iota(_i32, (1, NP), 1) * TB,
                     total_pad - 1)
    ind = (poff.astype(_i32) <= bt).astype(_i32)
    be_ref[...] = jnp.sum(ind, axis=0, keepdims=True) - 1
    meta_ref[...] = (total_pad // TB).reshape(1, 1)


@jax.jit
def _route(x, router_w):
    return pl.pallas_call(
        _route_body,
        out_shape=[
            jax.ShapeDtypeStruct((T, 128), _f32),
            jax.ShapeDtypeStruct((T, 128), _f32),
            jax.ShapeDtypeStruct((1, T), _i32),
            jax.ShapeDtypeStruct((1, T), _i32),
            jax.ShapeDtypeStruct((1, NP), _i32),
            jax.ShapeDtypeStruct((1, 1), _i32),
            jax.ShapeDtypeStruct((1, 1), _f32),
        ],
    )(x, router_w)


def _shared_body(x_ref, wi0_ref, wi1_ref, wo_ref, gw_ref, out_ref):
    x = x_ref[...]
    xb = x.astype(jnp.bfloat16)
    h0 = jnp.dot(xb, wi0_ref[...].astype(jnp.bfloat16),
                 preferred_element_type=_f32)
    h1 = jnp.dot(xb, wi1_ref[...].astype(jnp.bfloat16),
                 preferred_element_type=_f32)
    act = jax.nn.silu(h0) * h1
    shared = jnp.dot(act.astype(jnp.bfloat16), wo_ref[...].astype(jnp.bfloat16),
                     preferred_element_type=_f32)
    gate = jax.nn.sigmoid(jnp.dot(x, gw_ref[...], preferred_element_type=_f32))
    out_ref[...] = gate * shared


@jax.jit
def _shared(x, wi0, wi1, wo, gw):
    return pl.pallas_call(
        _shared_body,
        out_shape=jax.ShapeDtypeStruct((T, D), _f32),
    )(x, wi0, wi1, wo, gw)


def _dispatch_body(x_hbm, inv0_hbm, inv1_hbm, tw0_hbm, tw1_hbm,
                   xs_hbm, sw_hbm,
                   rows_v, idx0_v, idx1_v, wbuf_v, sem0, sem1):
    wid = lax.axis_index("s") * NC + lax.axis_index("c")
    base = wid * TPW
    pltpu.sync_copy(x_hbm.at[pl.ds(base, TPW), :], rows_v)
    pltpu.sync_copy(inv0_hbm.at[0, pl.ds(base, TPW)], idx0_v)
    pltpu.sync_copy(inv1_hbm.at[0, pl.ds(base, TPW)], idx1_v)
    c0 = pltpu.async_copy(rows_v, xs_hbm.at[idx0_v], sem0)
    c1 = pltpu.async_copy(rows_v, xs_hbm.at[idx1_v], sem1)
    c0.wait()
    c1.wait()
    for k in range(K):
        src = tw0_hbm if k == 0 else tw1_hbm
        idx = idx0_v if k == 0 else idx1_v
        pltpu.sync_copy(src.at[pl.ds(base, TPW), :], wbuf_v)
        c = pltpu.async_copy(wbuf_v, sw_hbm.at[idx], sem0)
        c.wait()


@jax.jit
def _dispatch(x, inv0, inv1, tw0, tw1):
    mesh = plsc.VectorSubcoreMesh(core_axis_name="c", subcore_axis_name="s")
    f = functools.partial(
        pl.kernel,
        out_type=(jax.ShapeDtypeStruct((NROWS, D), _f32),
                  jax.ShapeDtypeStruct((NROWS, 128), _f32)),
        mesh=mesh,
        scratch_types=[
            pltpu.VMEM((TPW, D), _f32),
            pltpu.VMEM((TPW,), _i32),
            pltpu.VMEM((TPW,), _i32),
            pltpu.VMEM((TPW, 128), _f32),
            pltpu.SemaphoreType.DMA,
            pltpu.SemaphoreType.DMA,
        ],
    )(_dispatch_body)
    return f(x, inv0, inv1, tw0, tw1)


def _gmm_body(be_ref, mt_ref, xs_ref, sw_ref, w0_ref, w1_ref, wo_ref, ys_ref):
    b = pl.program_id(0)

    @pl.when(b < mt_ref[0])
    def _():
        xb = xs_ref[...].astype(jnp.bfloat16)
        h0 = jnp.dot(xb, w0_ref[0].astype(jnp.bfloat16),
                     preferred_element_type=_f32)
        h1 = jnp.dot(xb, w1_ref[0].astype(jnp.bfloat16),
                     preferred_element_type=_f32)
        act = jax.nn.silu(h0) * h1
        y = jnp.dot(act.astype(jnp.bfloat16), wo_ref[0].astype(jnp.bfloat16),
                    preferred_element_type=_f32)
        ys_ref[...] = y * sw_ref[:, 0:1]


@jax.jit
def _gmm(be, meta, xs, sw, w0, w1, wo):
    grid_spec = pltpu.PrefetchScalarGridSpec(
        num_scalar_prefetch=2,
        grid=(NP,),
        in_specs=[
            pl.BlockSpec((TB, D), lambda b, be_r, mt_r: (b, 0)),
            pl.BlockSpec((TB, 128), lambda b, be_r, mt_r: (b, 0)),
            pl.BlockSpec((1, D, F), lambda b, be_r, mt_r: (be_r[b], 0, 0)),
            pl.BlockSpec((1, D, F), lambda b, be_r, mt_r: (be_r[b], 0, 0)),
            pl.BlockSpec((1, F, D), lambda b, be_r, mt_r: (be_r[b], 0, 0)),
        ],
        out_specs=pl.BlockSpec((TB, D), lambda b, be_r, mt_r: (b, 0)),
    )
    return pl.pallas_call(
        _gmm_body,
        grid_spec=grid_spec,
        out_shape=jax.ShapeDtypeStruct((NROWS, D), _f32),
    )(be, meta, xs, sw, w0, w1, wo)


_CH = TPW // 2


def _combine_body(init_hbm, ys_hbm, inv0_hbm, inv1_hbm, out_hbm,
                  acc_v, g1_v, idx0_v, idx1_v, sem0, sem1):
    wid = lax.axis_index("s") * NC + lax.axis_index("c")
    base = wid * TPW
    pltpu.sync_copy(inv0_hbm.at[0, pl.ds(base, TPW)], idx0_v)
    pltpu.sync_copy(inv1_hbm.at[0, pl.ds(base, TPW)], idx1_v)
    for h in range(2):
        hb = base + h * _CH
        hs = pl.ds(h * _CH, _CH)
        c0 = pltpu.async_copy(ys_hbm.at[idx0_v.at[hs]], acc_v, sem0)
        c1 = pltpu.async_copy(ys_hbm.at[idx1_v.at[hs]], g1_v, sem1)
        c0.wait()
        c1.wait()

        def tok(i, carry):
            for c in range(D // 16):
                s = pl.ds(c * 16, 16)
                acc_v[i, s] = acc_v[i, s] + g1_v[i, s]
            return carry

        lax.fori_loop(0, _CH, tok, 0)
        pltpu.async_copy(init_hbm.at[pl.ds(hb, _CH), :], g1_v, sem1).wait()
        lax.fori_loop(0, _CH, tok, 0)
        pltpu.sync_copy(acc_v, out_hbm.at[pl.ds(hb, _CH), :])


@jax.jit
def _combine(init, ys, inv0, inv1):
    mesh = plsc.VectorSubcoreMesh(core_axis_name="c", subcore_axis_name="s")
    f = functools.partial(
        pl.kernel,
        out_type=jax.ShapeDtypeStruct((T, D), _f32),
        mesh=mesh,
        scratch_types=[
            pltpu.VMEM((_CH, D), _f32),
            pltpu.VMEM((_CH, D), _f32),
            pltpu.VMEM((TPW,), _i32),
            pltpu.VMEM((TPW,), _i32),
            pltpu.SemaphoreType.DMA,
            pltpu.SemaphoreType.DMA,
        ],
    )(_combine_body)
    return f(init, ys, inv0, inv1)


def kernel(hidden_states, deterministic, router_w, w0, w1, wo, shared_wi0,
           shared_wi1, shared_wo, shared_gate_w):
    x = hidden_states.reshape(T, D)
    tw0, tw1, inv0, inv1, be, meta, loss = _route(x, router_w)
    init = _shared(x, shared_wi0, shared_wi1, shared_wo, shared_gate_w)
    xs, sw = _dispatch(x, inv0, inv1, tw0, tw1)
    ys = _gmm(be.reshape(NP), meta.reshape(1), xs, sw, w0, w1, wo)
    out = _combine(init, ys, inv0, inv1)
    return out.reshape(hidden_states.shape), loss[0, 0]


# speedup vs baseline: 1.0248x; 1.0248x over previous
import functools

import jax
import jax.numpy as jnp
from jax import lax
from jax.experimental import pallas as pl
from jax.experimental.pallas import tpu as pltpu
from jax.experimental.pallas import tpu_sc as plsc

B_, S, D, E, F, K = 1, 2048, 1024, 8, 512, 2
T = B_ * S
TB = 128
NP = T * K // TB + E
NROWS = NP * TB
NC, NS = 2, 16
NW = NC * NS
TPW = T // NW

_f32 = jnp.float32
_i32 = jnp.int32


def _route_body(x_ref, rw_ref, tw0_ref, tw1_ref, inv0_ref, inv1_ref,
                be_ref, meta_ref, loss_ref):
    x = x_ref[...]
    logits = jnp.dot(x, rw_ref[...], preferred_element_type=_f32)
    probs = jax.nn.softmax(logits, axis=-1)
    i1 = jnp.argmax(probs, axis=-1)
    m1 = jnp.max(probs, axis=-1, keepdims=True)
    iota_e = lax.broadcasted_iota(_i32, probs.shape, 1)
    oh1 = iota_e == i1[:, None]
    masked = jnp.where(oh1, -jnp.inf, probs)
    i2 = jnp.argmax(masked, axis=-1)
    m2 = jnp.max(masked, axis=-1, keepdims=True)
    oh2 = iota_e == i2[:, None]
    denom = m1 + m2
    tw0_ref[...] = jnp.broadcast_to(m1 / denom, (T, 128))
    tw1_ref[...] = jnp.broadcast_to(m2 / denom, (T, 128))
    mask = (oh1 | oh2).astype(_f32)
    loss_ref[...] = (E * jnp.sum(jnp.mean(mask, axis=0)
                                 * jnp.mean(probs, axis=0))).reshape(1, 1)

    maskT = jnp.transpose(mask)
    incl = maskT
    sh = 1
    while sh < T:
        incl = incl + jnp.concatenate(
            [jnp.zeros((E, sh), _f32), incl[:, :T - sh]], axis=1)
        sh *= 2
    posE = incl - maskT
    counts = incl[:, T - 1:T]

    pc = ((counts.astype(_i32) + (TB - 1)) // TB) * TB
    r8 = lax.broadcasted_iota(_i32, (E, E), 0)
    c8 = lax.broadcasted_iota(_i32, (E, E), 1)
    tri8 = (r8 > c8).astype(_f32)
    poff = jnp.dot(tri8, pc.astype(_f32), preferred_element_type=_f32)
    total_pad = jnp.sum(pc)

    inv_all = poff + posE
    ohT1 = jnp.transpose(oh1.astype(_f32))
    ohT2 = jnp.transpose(oh2.astype(_f32))
    inv0_ref[...] = jnp.sum(ohT1 * inv_all, axis=0, keepdims=True).astype(_i32)
    inv1_ref[...] = jnp.sum(ohT2 * inv_all, axis=0, keepdims=True).astype(_i32)

    bt = jnp.minimum(lax.broadcasted_iota(_i32, (1, NP), 1) * TB,
                     total_pad - 1)
    ind = (poff.astype(_i32) <= bt).astype(_i32)
    be_ref[...] = jnp.sum(ind, axis=0, keepdims=True) - 1
    meta_ref[...] = (total_pad // TB).reshape(1, 1)


@jax.jit
def _route(x, router_w):
    return pl.pallas_call(
        _route_body,
        out_shape=[
            jax.ShapeDtypeStruct((T, 128), _f32),
            jax.ShapeDtypeStruct((T, 128), _f32),
            jax.ShapeDtypeStruct((1, T), _i32),
            jax.ShapeDtypeStruct((1, T), _i32),
            jax.ShapeDtypeStruct((1, NP), _i32),
            jax.ShapeDtypeStruct((1, 1), _i32),
            jax.ShapeDtypeStruct((1, 1), _f32),
        ],
    )(x, router_w)


def _shared_body(x_ref, wi0_ref, wi1_ref, wo_ref, gw_ref, out_ref):
    x = x_ref[...]
    xb = x.astype(jnp.bfloat16)
    h0 = jnp.dot(xb, wi0_ref[...].astype(jnp.bfloat16),
                 preferred_element_type=_f32)
    h1 = jnp.dot(xb, wi1_ref[...].astype(jnp.bfloat16),
                 preferred_element_type=_f32)
    act = jax.nn.silu(h0) * h1
    shared = jnp.dot(act.astype(jnp.bfloat16), wo_ref[...].astype(jnp.bfloat16),
                     preferred_element_type=_f32)
    gate = jax.nn.sigmoid(jnp.dot(x, gw_ref[...], preferred_element_type=_f32))
    out_ref[...] = gate * shared


_STB = T // 4


@jax.jit
def _shared(x, wi0, wi1, wo, gw):
    return pl.pallas_call(
        _shared_body,
        grid=(T // _STB,),
        in_specs=[
            pl.BlockSpec((_STB, D), lambda i: (i, 0)),
            pl.BlockSpec((D, F), lambda i: (0, 0)),
            pl.BlockSpec((D, F), lambda i: (0, 0)),
            pl.BlockSpec((F, D), lambda i: (0, 0)),
            pl.BlockSpec((D, 1), lambda i: (0, 0)),
        ],
        out_specs=pl.BlockSpec((_STB, D), lambda i: (i, 0)),
        out_shape=jax.ShapeDtypeStruct((T, D), _f32),
    )(x, wi0, wi1, wo, gw)


def _dispatch_body(x_hbm, inv0_hbm, inv1_hbm, tw0_hbm, tw1_hbm,
                   xs_hbm, sw_hbm,
                   rows_v, idx0_v, idx1_v, wbuf_v, sem0, sem1):
    wid = lax.axis_index("s") * NC + lax.axis_index("c")
    base = wid * TPW
    pltpu.sync_copy(x_hbm.at[pl.ds(base, TPW), :], rows_v)
    pltpu.sync_copy(inv0_hbm.at[0, pl.ds(base, TPW)], idx0_v)
    pltpu.sync_copy(inv1_hbm.at[0, pl.ds(base, TPW)], idx1_v)
    c0 = pltpu.async_copy(rows_v, xs_hbm.at[idx0_v], sem0)
    c1 = pltpu.async_copy(rows_v, xs_hbm.at[idx1_v], sem1)
    c0.wait()
    c1.wait()
    for k in range(K):
        src = tw0_hbm if k == 0 else tw1_hbm
        idx = idx0_v if k == 0 else idx1_v
        pltpu.sync_copy(src.at[pl.ds(base, TPW), :], wbuf_v)
        c = pltpu.async_copy(wbuf_v, sw_hbm.at[idx], sem0)
        c.wait()


@jax.jit
def _dispatch(x, inv0, inv1, tw0, tw1):
    mesh = plsc.VectorSubcoreMesh(core_axis_name="c", subcore_axis_name="s")
    f = functools.partial(
        pl.kernel,
        out_type=(jax.ShapeDtypeStruct((NROWS, D), _f32),
                  jax.ShapeDtypeStruct((NROWS, 128), _f32)),
        mesh=mesh,
        scratch_types=[
            pltpu.VMEM((TPW, D), _f32),
            pltpu.VMEM((TPW,), _i32),
            pltpu.VMEM((TPW,), _i32),
            pltpu.VMEM((TPW, 128), _f32),
            pltpu.SemaphoreType.DMA,
            pltpu.SemaphoreType.DMA,
        ],
    )(_dispatch_body)
    return f(x, inv0, inv1, tw0, tw1)


def _gmm_body(be_ref, mt_ref, xs_ref, sw_ref, w0_ref, w1_ref, wo_ref, ys_ref):
    b = pl.program_id(0)

    @pl.when(b < mt_ref[0])
    def _():
        xb = xs_ref[...].astype(jnp.bfloat16)
        h0 = jnp.dot(xb, w0_ref[0].astype(jnp.bfloat16),
                     preferred_element_type=_f32)
        h1 = jnp.dot(xb, w1_ref[0].astype(jnp.bfloat16),
                     preferred_element_type=_f32)
        act = jax.nn.silu(h0) * h1
        y = jnp.dot(act.astype(jnp.bfloat16), wo_ref[0].astype(jnp.bfloat16),
                    preferred_element_type=_f32)
        ys_ref[...] = y * sw_ref[:, 0:1]


@jax.jit
def _gmm(be, meta, xs, sw, w0, w1, wo):
    grid_spec = pltpu.PrefetchScalarGridSpec(
        num_scalar_prefetch=2,
        grid=(NP,),
        in_specs=[
            pl.BlockSpec((TB, D), lambda b, be_r, mt_r: (b, 0)),
            pl.BlockSpec((TB, 128), lambda b, be_r, mt_r: (b, 0)),
            pl.BlockSpec((1, D, F), lambda b, be_r, mt_r: (be_r[b], 0, 0)),
            pl.BlockSpec((1, D, F), lambda b, be_r, mt_r: (be_r[b], 0, 0)),
            pl.BlockSpec((1, F, D), lambda b, be_r, mt_r: (be_r[b], 0, 0)),
        ],
        out_specs=pl.BlockSpec((TB, D), lambda b, be_r, mt_r: (b, 0)),
    )
    return pl.pallas_call(
        _gmm_body,
        grid_spec=grid_spec,
        out_shape=jax.ShapeDtypeStruct((NROWS, D), _f32),
    )(be, meta, xs, sw, w0, w1, wo)


_CH = TPW // 2


def _combine_body(init_hbm, ys_hbm, inv0_hbm, inv1_hbm, out_hbm,
                  acc_v, g1_v, init_v, idx0_v, idx1_v, sem0, sem1, sem2):
    wid = lax.axis_index("s") * NC + lax.axis_index("c")
    base = wid * TPW
    pltpu.sync_copy(inv0_hbm.at[0, pl.ds(base, TPW)], idx0_v)
    pltpu.sync_copy(inv1_hbm.at[0, pl.ds(base, TPW)], idx1_v)
    for h in range(2):
        hb = base + h * _CH
        hs = pl.ds(h * _CH, _CH)
        c0 = pltpu.async_copy(ys_hbm.at[idx0_v.at[hs]], acc_v, sem0)
        c1 = pltpu.async_copy(ys_hbm.at[idx1_v.at[hs]], g1_v, sem1)
        c2 = pltpu.async_copy(init_hbm.at[pl.ds(hb, _CH), :], init_v, sem2)
        c0.wait()
        c1.wait()
        c2.wait()

        def tok(i, carry):
            for c in range(D // 16):
                s = pl.ds(c * 16, 16)
                acc_v[i, s] = acc_v[i, s] + g1_v[i, s] + init_v[i, s]
            return carry

        lax.fori_loop(0, _CH, tok, 0)
        pltpu.sync_copy(acc_v, out_hbm.at[pl.ds(hb, _CH), :])


@jax.jit
def _combine(init, ys, inv0, inv1):
    mesh = plsc.VectorSubcoreMesh(core_axis_name="c", subcore_axis_name="s")
    f = functools.partial(
        pl.kernel,
        out_type=jax.ShapeDtypeStruct((T, D), _f32),
        mesh=mesh,
        scratch_types=[
            pltpu.VMEM((_CH, D), _f32),
            pltpu.VMEM((_CH, D), _f32),
            pltpu.VMEM((_CH, D), _f32),
            pltpu.VMEM((TPW,), _i32),
            pltpu.VMEM((TPW,), _i32),
            pltpu.SemaphoreType.DMA,
            pltpu.SemaphoreType.DMA,
            pltpu.SemaphoreType.DMA,
        ],
    )(_combine_body)
    return f(init, ys, inv0, inv1)


def kernel(hidden_states, deterministic, router_w, w0, w1, wo, shared_wi0,
           shared_wi1, shared_wo, shared_gate_w):
    x = hidden_states.reshape(T, D)
    tw0, tw1, inv0, inv1, be, meta, loss = _route(x, router_w)
    init = _shared(x, shared_wi0, shared_wi1, shared_wo, shared_gate_w)
    xs, sw = _dispatch(x, inv0, inv1, tw0, tw1)
    ys = _gmm(be.reshape(NP), meta.reshape(1), xs, sw, w0, w1, wo)
    out = _combine(init, ys, inv0, inv1)
    return out.reshape(hidden_states.shape), loss[0, 0]


# speedup vs baseline: 1.0454x; 1.0201x over previous
import functools

import jax
import jax.numpy as jnp
from jax import lax
from jax.experimental import pallas as pl
from jax.experimental.pallas import tpu as pltpu
from jax.experimental.pallas import tpu_sc as plsc

B_, S, D, E, F, K = 1, 2048, 1024, 8, 512, 2
T = B_ * S
TB = 128
NP = T * K // TB + E
NROWS = NP * TB
NC, NS = 2, 16
NW = NC * NS
TPW = T // NW

_f32 = jnp.float32
_i32 = jnp.int32


def _route_body(x_ref, rw_ref, tw0_ref, tw1_ref, inv0_ref, inv1_ref,
                be_ref, meta_ref, loss_ref):
    x = x_ref[...]
    logits = jnp.dot(x, rw_ref[...], preferred_element_type=_f32)
    probs = jax.nn.softmax(logits, axis=-1)
    i1 = jnp.argmax(probs, axis=-1)
    m1 = jnp.max(probs, axis=-1, keepdims=True)
    iota_e = lax.broadcasted_iota(_i32, probs.shape, 1)
    oh1 = iota_e == i1[:, None]
    masked = jnp.where(oh1, -jnp.inf, probs)
    i2 = jnp.argmax(masked, axis=-1)
    m2 = jnp.max(masked, axis=-1, keepdims=True)
    oh2 = iota_e == i2[:, None]
    denom = m1 + m2
    tw0_ref[...] = jnp.broadcast_to(m1 / denom, (T, 128))
    tw1_ref[...] = jnp.broadcast_to(m2 / denom, (T, 128))
    mask = (oh1 | oh2).astype(_f32)
    loss_ref[...] = (E * jnp.sum(jnp.mean(mask, axis=0)
                                 * jnp.mean(probs, axis=0))).reshape(1, 1)

    maskT = jnp.transpose(mask)
    incl = maskT
    sh = 1
    while sh < T:
        incl = incl + jnp.concatenate(
            [jnp.zeros((E, sh), _f32), incl[:, :T - sh]], axis=1)
        sh *= 2
    posE = incl - maskT
    counts = incl[:, T - 1:T]

    pc = ((counts.astype(_i32) + (TB - 1)) // TB) * TB
    r8 = lax.broadcasted_iota(_i32, (E, E), 0)
    c8 = lax.broadcasted_iota(_i32, (E, E), 1)
    tri8 = (r8 > c8).astype(_f32)
    poff = jnp.dot(tri8, pc.astype(_f32), preferred_element_type=_f32)
    total_pad = jnp.sum(pc)

    inv_all = poff + posE
    ohT1 = jnp.transpose(oh1.astype(_f32))
    ohT2 = jnp.transpose(oh2.astype(_f32))
    inv0_ref[...] = jnp.sum(ohT1 * inv_all, axis=0, keepdims=True).astype(_i32)
    inv1_ref[...] = jnp.sum(ohT2 * inv_all, axis=0, keepdims=True).astype(_i32)

    bt = jnp.minimum(lax.broadcasted_iota(_i32, (1, NP), 1) * TB,
                     total_pad - 1)
    ind = (poff.astype(_i32) <= bt).astype(_i32)
    be_ref[...] = jnp.sum(ind, axis=0, keepdims=True) - 1
    meta_ref[...] = (total_pad // TB).reshape(1, 1)


@jax.jit
def _route(x, router_w):
    return pl.pallas_call(
        _route_body,
        out_shape=[
            jax.ShapeDtypeStruct((T, 128), _f32),
            jax.ShapeDtypeStruct((T, 128), _f32),
            jax.ShapeDtypeStruct((1, T), _i32),
            jax.ShapeDtypeStruct((1, T), _i32),
            jax.ShapeDtypeStruct((1, NP), _i32),
            jax.ShapeDtypeStruct((1, 1), _i32),
            jax.ShapeDtypeStruct((1, 1), _f32),
        ],
    )(x, router_w)


def _shared_body(x_ref, wi0_ref, wi1_ref, wo_ref, gw_ref, out_ref):
    x = x_ref[...]
    xb = x.astype(jnp.bfloat16)
    h0 = jnp.dot(xb, wi0_ref[...].astype(jnp.bfloat16),
                 preferred_element_type=_f32)
    h1 = jnp.dot(xb, wi1_ref[...].astype(jnp.bfloat16),
                 preferred_element_type=_f32)
    act = jax.nn.silu(h0) * h1
    shared = jnp.dot(act.astype(jnp.bfloat16), wo_ref[...].astype(jnp.bfloat16),
                     preferred_element_type=_f32)
    gate = jax.nn.sigmoid(jnp.dot(x, gw_ref[...], preferred_element_type=_f32))
    out_ref[...] = gate * shared


_STB = T // 4


@jax.jit
def _shared(x, wi0, wi1, wo, gw):
    return pl.pallas_call(
        _shared_body,
        grid=(T // _STB,),
        in_specs=[
            pl.BlockSpec((_STB, D), lambda i: (i, 0)),
            pl.BlockSpec((D, F), lambda i: (0, 0)),
            pl.BlockSpec((D, F), lambda i: (0, 0)),
            pl.BlockSpec((F, D), lambda i: (0, 0)),
            pl.BlockSpec((D, 1), lambda i: (0, 0)),
        ],
        out_specs=pl.BlockSpec((_STB, D), lambda i: (i, 0)),
        out_shape=jax.ShapeDtypeStruct((T, D), _f32),
    )(x, wi0, wi1, wo, gw)


def _dispatch_body(x_hbm, inv0_hbm, inv1_hbm, tw0_hbm, tw1_hbm,
                   xs_hbm, sw_hbm,
                   rows_v, idx0_v, idx1_v, wbuf_v, sem0, sem1):
    wid = lax.axis_index("s") * NC + lax.axis_index("c")
    base = wid * TPW
    pltpu.sync_copy(x_hbm.at[pl.ds(base, TPW), :], rows_v)
    pltpu.sync_copy(inv0_hbm.at[0, pl.ds(base, TPW)], idx0_v)
    pltpu.sync_copy(inv1_hbm.at[0, pl.ds(base, TPW)], idx1_v)
    c0 = pltpu.async_copy(rows_v, xs_hbm.at[idx0_v], sem0)
    c1 = pltpu.async_copy(rows_v, xs_hbm.at[idx1_v], sem1)
    c0.wait()
    c1.wait()
    for k in range(K):
        src = tw0_hbm if k == 0 else tw1_hbm
        idx = idx0_v if k == 0 else idx1_v
        pltpu.sync_copy(src.at[pl.ds(base, TPW), :], wbuf_v)
        c = pltpu.async_copy(wbuf_v, sw_hbm.at[idx], sem0)
        c.wait()


@jax.jit
def _dispatch(x, inv0, inv1, tw0, tw1):
    mesh = plsc.VectorSubcoreMesh(core_axis_name="c", subcore_axis_name="s")
    f = functools.partial(
        pl.kernel,
        out_type=(jax.ShapeDtypeStruct((NROWS, D), _f32),
                  jax.ShapeDtypeStruct((NROWS, 128), _f32)),
        mesh=mesh,
        scratch_types=[
            pltpu.VMEM((TPW, D), _f32),
            pltpu.VMEM((TPW,), _i32),
            pltpu.VMEM((TPW,), _i32),
            pltpu.VMEM((TPW, 128), _f32),
            pltpu.SemaphoreType.DMA,
            pltpu.SemaphoreType.DMA,
        ],
    )(_dispatch_body)
    return f(x, inv0, inv1, tw0, tw1)


def _gmm_body(be_ref, mt_ref, xs_ref, sw_ref, w0_ref, w1_ref, wo_ref, ys_ref):
    b = pl.program_id(0)

    @pl.when(b < mt_ref[0])
    def _():
        xb = xs_ref[...].astype(jnp.bfloat16)
        h0 = jnp.dot(xb, w0_ref[0].astype(jnp.bfloat16),
                     preferred_element_type=_f32)
        h1 = jnp.dot(xb, w1_ref[0].astype(jnp.bfloat16),
                     preferred_element_type=_f32)
        act = jax.nn.silu(h0) * h1
        y = jnp.dot(act.astype(jnp.bfloat16), wo_ref[0].astype(jnp.bfloat16),
                    preferred_element_type=_f32)
        ys_ref[...] = y * sw_ref[:, 0:1]


@jax.jit
def _gmm(be, meta, xs, sw, w0, w1, wo):
    grid_spec = pltpu.PrefetchScalarGridSpec(
        num_scalar_prefetch=2,
        grid=(NP,),
        in_specs=[
            pl.BlockSpec((TB, D), lambda b, be_r, mt_r: (b, 0)),
            pl.BlockSpec((TB, 128), lambda b, be_r, mt_r: (b, 0)),
            pl.BlockSpec((1, D, F), lambda b, be_r, mt_r: (be_r[b], 0, 0)),
            pl.BlockSpec((1, D, F), lambda b, be_r, mt_r: (be_r[b], 0, 0)),
            pl.BlockSpec((1, F, D), lambda b, be_r, mt_r: (be_r[b], 0, 0)),
        ],
        out_specs=pl.BlockSpec((TB, D), lambda b, be_r, mt_r: (b, 0)),
    )
    return pl.pallas_call(
        _gmm_body,
        grid_spec=grid_spec,
        out_shape=jax.ShapeDtypeStruct((NROWS, D), _f32),
    )(be, meta, xs, sw, w0, w1, wo)


_NQ = 4
_CH = TPW // _NQ


def _combine_body(init_hbm, ys_hbm, inv0_hbm, inv1_hbm, out_hbm,
                  acc_v, g1_v, init_v, idx0_v, idx1_v, sems):
    wid = lax.axis_index("s") * NC + lax.axis_index("c")
    base = wid * TPW
    pltpu.sync_copy(inv0_hbm.at[0, pl.ds(base, TPW)], idx0_v)
    pltpu.sync_copy(inv1_hbm.at[0, pl.ds(base, TPW)], idx1_v)

    def start(q, buf):
        hs = pl.ds(q * _CH, _CH)
        return (
            pltpu.async_copy(ys_hbm.at[idx0_v.at[hs]], acc_v.at[buf],
                             sems.at[buf, 0]),
            pltpu.async_copy(ys_hbm.at[idx1_v.at[hs]], g1_v.at[buf],
                             sems.at[buf, 1]),
            pltpu.async_copy(init_hbm.at[pl.ds(base + q * _CH, _CH), :],
                             init_v.at[buf], sems.at[buf, 2]),
        )

    pend = start(0, 0)
    for q in range(_NQ):
        cur = q % 2
        cs = pend
        if q + 1 < _NQ:
            pend = start(q + 1, (q + 1) % 2)
        for c_ in cs:
            c_.wait()

        def tok(i, carry):
            for c in range(D // 16):
                s = pl.ds(c * 16, 16)
                acc_v[cur, i, s] = (acc_v[cur, i, s] + g1_v[cur, i, s]
                                    + init_v[cur, i, s])
            return carry

        lax.fori_loop(0, _CH, tok, 0)
        pltpu.sync_copy(acc_v.at[cur], out_hbm.at[pl.ds(base + q * _CH, _CH), :])


@jax.jit
def _combine(init, ys, inv0, inv1):
    mesh = plsc.VectorSubcoreMesh(core_axis_name="c", subcore_axis_name="s")
    f = functools.partial(
        pl.kernel,
        out_type=jax.ShapeDtypeStruct((T, D), _f32),
        mesh=mesh,
        scratch_types=[
            pltpu.VMEM((2, _CH, D), _f32),
            pltpu.VMEM((2, _CH, D), _f32),
            pltpu.VMEM((2, _CH, D), _f32),
            pltpu.VMEM((TPW,), _i32),
            pltpu.VMEM((TPW,), _i32),
            pltpu.SemaphoreType.DMA((2, 3)),
        ],
    )(_combine_body)
    return f(init, ys, inv0, inv1)


def kernel(hidden_states, deterministic, router_w, w0, w1, wo, shared_wi0,
           shared_wi1, shared_wo, shared_gate_w):
    x = hidden_states.reshape(T, D)
    tw0, tw1, inv0, inv1, be, meta, loss = _route(x, router_w)
    init = _shared(x, shared_wi0, shared_wi1, shared_wo, shared_gate_w)
    xs, sw = _dispatch(x, inv0, inv1, tw0, tw1)
    ys = _gmm(be.reshape(NP), meta.reshape(1), xs, sw, w0, w1, wo)
    out = _combine(init, ys, inv0, inv1)
    return out.reshape(hidden_states.shape), loss[0, 0]
